# Initial kernel scaffold; baseline (speedup 1.0000x reference)
import jax
import jax.numpy as jnp
from jax.experimental import pallas as pl


def kernel(x, edge_index1, edge_index2, edge_index3, W1_1, b1_1, W2_1, b2_1, W3_1, b3_1, W1_2, b1_2, W2_2, b2_2, W3_2, b3_2, g1, be1, g2, be2, g3, be3):
    raise NotImplementedError("write your pallas kernel here")


# speedup vs baseline: 2.6944x; 2.6944x over previous
import functools

import jax
import jax.numpy as jnp
from jax import lax
from jax.experimental import pallas as pl
from jax.experimental.pallas import tpu as pltpu
from jax.experimental.pallas import tpu_sc as plsc

N = 10000
E = 320000
D_IN = 128
D_H = 128
D_OUT = 64

NC = 2
NS = 16
CHUNK = 80
EPT = E // NS
NCHUNK = EPT // CHUNK
ROWS_PT = N // NS
DEG_PT = 624
DEG_TAIL = N - NS * DEG_PT

_EPS = 1e-5


def _bn1_body(x_ref, g_ref, be_ref, out_ref):
    x = x_ref[...]
    mu = jnp.mean(x, axis=0, keepdims=True)
    xc = x - mu
    var = jnp.mean(xc * xc, axis=0, keepdims=True)
    hn = g_ref[...] * (xc * lax.rsqrt(var + _EPS)) + be_ref[...]
    out_ref[0:N, :] = hn[:, 0:64]
    out_ref[N:2 * N, :] = hn[:, 64:128]


RB = 1000
GRID = N // RB


def _mid1_body(acc_ref, deg_ref, w1_ref, b1_ref, h1_ref, st_ref):
    i = pl.program_id(0)
    h1 = None
    for r in range(3):
        hr = jnp.concatenate([acc_ref[r, 0], acc_ref[r, 1]], axis=1)
        hr = hr * (1.0 / (deg_ref[r] + 1.0))
        t = jnp.dot(hr, w1_ref[r], preferred_element_type=jnp.float32)
        t = t + b1_ref[r]
        h1 = t if h1 is None else h1 + t
    h1 = jnp.maximum(h1, 0.0)
    h1_ref[...] = h1
    part = jnp.concatenate(
        [jnp.sum(h1, axis=0, keepdims=True),
         jnp.sum(h1 * h1, axis=0, keepdims=True)], axis=0)

    @pl.when(i == 0)
    def _():
        st_ref[...] = part

    @pl.when(i != 0)
    def _():
        st_ref[...] = st_ref[...] + part


def _mid2_body(h1_ref, st_ref, w2_ref, g_ref, be_ref, out_ref):
    mu = st_ref[0:1, :] * (1.0 / N)
    var = st_ref[1:2, :] * (1.0 / N) - mu * mu
    hb = g_ref[...] * ((h1_ref[...] - mu) * lax.rsqrt(var + _EPS)) + be_ref[...]
    for r in range(3):
        y = jnp.dot(hb, w2_ref[r], preferred_element_type=jnp.float32)
        out_ref[r, 0] = y[:, 0:32]
        out_ref[r, 1] = y[:, 32:64]


def _fin1_body(acc_ref, deg_ref, b2_ref, h2_ref, st_ref):
    i = pl.program_id(0)
    h2 = None
    for r in range(3):
        hr = jnp.concatenate([acc_ref[r, 0], acc_ref[r, 1]], axis=1)
        t = hr * (1.0 / (deg_ref[r] + 1.0)) + b2_ref[r]
        h2 = t if h2 is None else h2 + t
    h2_ref[...] = h2
    part = jnp.concatenate(
        [jnp.sum(h2, axis=0, keepdims=True),
         jnp.sum(h2 * h2, axis=0, keepdims=True)], axis=0)

    @pl.when(i == 0)
    def _():
        st_ref[...] = part

    @pl.when(i != 0)
    def _():
        st_ref[...] = st_ref[...] + part


def _fin2_body(h2_ref, st_ref, g_ref, be_ref, out_ref):
    mu = st_ref[0:1, :] * (1.0 / N)
    var = st_ref[1:2, :] * (1.0 / N) - mu * mu
    hn = g_ref[...] * ((h2_ref[...] - mu) * lax.rsqrt(var + _EPS)) + be_ref[...]
    sg = 1.0 / (1.0 + jnp.exp(-hn))
    zmax = jnp.max(sg, axis=1, keepdims=True)
    zmin = jnp.min(sg, axis=1, keepdims=True)
    out = (sg - zmin) / (zmax - zmin)
    nrm = jnp.sqrt(jnp.sum(out * out, axis=1, keepdims=True))
    out_ref[...] = out / jnp.maximum(nrm, 1e-12)


def _sc_agg_body(width, with_deg, rel_stride, table_ref, e1_ref, e2_ref,
                 e3_ref, zn_ref, *rest):
    if with_deg:
        (acc_out, deg_out, a0, a1, a2, d0, d1, d2,
         idx_v, dst_v, rows_v, ones_v, sem) = rest
        accs = (a0, a1, a2)
        degs = (d0, d1, d2)
    else:
        (acc_out, a0, a1, a2, idx_v, dst_v, rows_v, sem) = rest
        accs = (a0, a1, a2)
        degs = None

    c = lax.axis_index("c")
    t = lax.axis_index("s")
    row0 = t * ROWS_PT
    cN = c * N

    for r in range(3):
        pltpu.sync_copy(table_ref.at[pl.ds(r * rel_stride + cN + row0,
                                           ROWS_PT)],
                        accs[r].at[pl.ds(row0, ROWS_PT)])
    if with_deg:
        @pl.when(c == 0)
        def _():
            for r in range(3):
                pltpu.sync_copy(zn_ref.at[pl.ds(t * DEG_PT, DEG_PT)],
                                degs[r].at[pl.ds(t * DEG_PT, DEG_PT)])

            @pl.when(t == 0)
            def _():
                for r in range(3):
                    pltpu.sync_copy(zn_ref.at[pl.ds(NS * DEG_PT, DEG_TAIL)],
                                    degs[r].at[pl.ds(NS * DEG_PT, DEG_TAIL)])
        for j in range(CHUNK // 16):
            ones_v[pl.ds(j * 16, 16)] = jnp.ones((16,), jnp.float32)

    plsc.subcore_barrier()

    base = t * EPT
    for r, e_ref in enumerate((e1_ref, e2_ref, e3_ref)):
        bias = r * rel_stride + cN

        def chunk_body(i, _, e_ref=e_ref, acc=accs[r],
                       deg=degs[r] if with_deg else None, bias=bias):
            off = base + i * CHUNK
            pltpu.sync_copy(e_ref.at[0, pl.ds(off, CHUNK)], idx_v)
            for j in range(CHUNK // 16):
                sl = pl.ds(j * 16, 16)
                idx_v[sl] = idx_v[sl] + bias
            pltpu.async_copy(table_ref.at[idx_v], rows_v, sem).wait()
            pltpu.sync_copy(e_ref.at[1, pl.ds(off, CHUNK)], dst_v)
            pltpu.sync_copy(rows_v, acc.at[dst_v], add=True)
            if with_deg:
                @pl.when(c == 0)
                def _():
                    pltpu.sync_copy(ones_v, deg.at[dst_v], add=True)
            return 0

        lax.fori_loop(0, NCHUNK, chunk_body, 0)

    plsc.subcore_barrier()

    for r in range(3):
        pltpu.sync_copy(accs[r].at[pl.ds(row0, ROWS_PT)],
                        acc_out.at[r, c, pl.ds(row0, ROWS_PT)])
    if with_deg:
        @pl.when(c == 0)
        def _():
            for r in range(3):
                pltpu.sync_copy(degs[r].at[pl.ds(t * DEG_PT, DEG_PT)],
                                deg_out.at[r, pl.ds(t * DEG_PT, DEG_PT)])

            @pl.when(t == 0)
            def _():
                for r in range(3):
                    pltpu.sync_copy(degs[r].at[pl.ds(NS * DEG_PT, DEG_TAIL)],
                                    deg_out.at[r, pl.ds(NS * DEG_PT, DEG_TAIL)])


def _make_sc_agg(width, with_deg, rel_stride):
    mesh = plsc.VectorSubcoreMesh(core_axis_name="c", subcore_axis_name="s",
                                  num_cores=NC, num_subcores=NS)
    out_type = [jax.ShapeDtypeStruct((3, NC, N, width), jnp.float32)]
    scratch = [
        pltpu.VMEM_SHARED((N, width), jnp.float32),
        pltpu.VMEM_SHARED((N, width), jnp.float32),
        pltpu.VMEM_SHARED((N, width), jnp.float32),
    ]
    if with_deg:
        out_type.append(jax.ShapeDtypeStruct((3, N), jnp.float32))
        scratch += [
            pltpu.VMEM_SHARED((N,), jnp.float32),
            pltpu.VMEM_SHARED((N,), jnp.float32),
            pltpu.VMEM_SHARED((N,), jnp.float32),
        ]
    scratch += [
        pltpu.VMEM((CHUNK,), jnp.int32),
        pltpu.VMEM((CHUNK,), jnp.int32),
        pltpu.VMEM((CHUNK, width), jnp.float32),
    ]
    if with_deg:
        scratch.append(pltpu.VMEM((CHUNK,), jnp.float32))
    scratch.append(pltpu.SemaphoreType.DMA)
    return pl.kernel(
        functools.partial(_sc_agg_body, width, with_deg, rel_stride),
        out_type=out_type,
        mesh=mesh,
        scratch_types=scratch,
        compiler_params=pltpu.CompilerParams(use_tc_tiling_on_sc=False),
    )


def kernel(x, edge_index1, edge_index2, edge_index3,
           W1_1, b1_1, W2_1, b2_1, W3_1, b3_1,
           W1_2, b1_2, W2_2, b2_2, W3_2, b3_2,
           g1, be1, g2, be2, g3, be3):
    h_split = pl.pallas_call(
        _bn1_body,
        out_shape=jax.ShapeDtypeStruct((2 * N, 64), jnp.float32),
    )(x, g1.reshape(1, D_IN), be1.reshape(1, D_IN))

    zeros_n = jnp.zeros((N,), jnp.float32)
    acc1, deg = _make_sc_agg(64, True, 0)(
        h_split, edge_index1, edge_index2, edge_index3, zeros_n)

    deg3 = deg.reshape(3, N, 1)
    w1 = jnp.stack([W1_1, W2_1, W3_1])
    b1 = jnp.stack([b1_1, b2_1, b3_1]).reshape(3, 1, D_H)
    w2 = jnp.stack([W1_2, W2_2, W3_2])

    acc_spec = lambda w: pl.BlockSpec((3, 2, RB, w), lambda i: (0, 0, i, 0))
    deg_spec = pl.BlockSpec((3, RB, 1), lambda i: (0, i, 0))
    full = lambda *s: pl.BlockSpec(s, lambda i: (0,) * len(s))
    row_spec = lambda w: pl.BlockSpec((RB, w), lambda i: (i, 0))

    h1, st1 = pl.pallas_call(
        _mid1_body,
        grid=(GRID,),
        in_specs=[acc_spec(64), deg_spec, full(3, D_IN, D_H),
                  full(3, 1, D_H)],
        out_specs=[row_spec(D_H), full(2, D_H)],
        out_shape=[jax.ShapeDtypeStruct((N, D_H), jnp.float32),
                   jax.ShapeDtypeStruct((2, D_H), jnp.float32)],
    )(acc1, deg3, w1, b1)

    table2 = pl.pallas_call(
        _mid2_body,
        grid=(GRID,),
        in_specs=[row_spec(D_H), full(2, D_H), full(3, D_H, D_OUT),
                  full(1, D_H), full(1, D_H)],
        out_specs=pl.BlockSpec((3, 2, RB, 32), lambda i: (0, 0, i, 0)),
        out_shape=jax.ShapeDtypeStruct((3, 2, N, 32), jnp.float32),
    )(h1, st1, w2, g2.reshape(1, D_H), be2.reshape(1, D_H))

    acc2 = _make_sc_agg(32, False, 2 * N)(
        table2.reshape(3 * 2 * N, 32),
        edge_index1, edge_index2, edge_index3, zeros_n)[0]

    b2 = jnp.stack([b1_2, b2_2, b3_2]).reshape(3, 1, D_OUT)
    h2, st2 = pl.pallas_call(
        _fin1_body,
        grid=(GRID,),
        in_specs=[acc_spec(32), deg_spec, full(3, 1, D_OUT)],
        out_specs=[row_spec(D_OUT), full(2, D_OUT)],
        out_shape=[jax.ShapeDtypeStruct((N, D_OUT), jnp.float32),
                   jax.ShapeDtypeStruct((2, D_OUT), jnp.float32)],
    )(acc2, deg3, b2)

    out = pl.pallas_call(
        _fin2_body,
        grid=(GRID,),
        in_specs=[row_spec(D_OUT), full(2, D_OUT), full(1, D_OUT),
                  full(1, D_OUT)],
        out_specs=row_spec(D_OUT),
        out_shape=jax.ShapeDtypeStruct((N, D_OUT), jnp.float32),
    )(h2, st2, g3.reshape(1, D_OUT), be3.reshape(1, D_OUT))
    return out


# speedup vs baseline: 8.8189x; 3.2731x over previous
import functools

import jax
import jax.numpy as jnp
from jax import lax
from jax.experimental import pallas as pl
from jax.experimental.pallas import tpu as pltpu
from jax.experimental.pallas import tpu_sc as plsc

N = 10000
E = 320000
D_IN = 128
D_H = 128
D_OUT = 64

NC = 2
NS = 16
SUB = 100
NBLK = E // SUB
NBT = NBLK // NS
ROWS_PT = N // NS
DEG_PT = 624
DEG_TAIL = N - NS * DEG_PT

_EPS = 1e-5


def _bn1_body(x_ref, g_ref, be_ref, out_ref):
    x = x_ref[...]
    mu = jnp.mean(x, axis=0, keepdims=True)
    xc = x - mu
    var = jnp.mean(xc * xc, axis=0, keepdims=True)
    hn = g_ref[...] * (xc * lax.rsqrt(var + _EPS)) + be_ref[...]
    out_ref[0:N, :] = hn[:, 0:64]
    out_ref[N:2 * N, :] = hn[:, 64:128]


RB = 1000
GRID = N // RB


def _mid1_body(acc_ref, deg_ref, w1_ref, b1_ref, h1_ref, st_ref):
    i = pl.program_id(0)
    h1 = None
    for r in range(3):
        hr = jnp.concatenate([acc_ref[r, 0], acc_ref[r, 1]], axis=1)
        hr = hr * (1.0 / (deg_ref[r] + 1.0))
        t = jnp.dot(hr, w1_ref[r], preferred_element_type=jnp.float32)
        t = t + b1_ref[r]
        h1 = t if h1 is None else h1 + t
    h1 = jnp.maximum(h1, 0.0)
    h1_ref[...] = h1
    part = jnp.concatenate(
        [jnp.sum(h1, axis=0, keepdims=True),
         jnp.sum(h1 * h1, axis=0, keepdims=True)], axis=0)

    @pl.when(i == 0)
    def _():
        st_ref[...] = part

    @pl.when(i != 0)
    def _():
        st_ref[...] = st_ref[...] + part


def _mid2_body(h1_ref, st_ref, w2_ref, g_ref, be_ref, out_ref):
    mu = st_ref[0:1, :] * (1.0 / N)
    var = st_ref[1:2, :] * (1.0 / N) - mu * mu
    hb = g_ref[...] * ((h1_ref[...] - mu) * lax.rsqrt(var + _EPS)) + be_ref[...]
    for r in range(3):
        y = jnp.dot(hb, w2_ref[r], preferred_element_type=jnp.float32)
        out_ref[r, 0] = y[:, 0:32]
        out_ref[r, 1] = y[:, 32:64]


def _fin1_body(acc_ref, deg_ref, b2_ref, h2_ref, st_ref):
    i = pl.program_id(0)
    h2 = None
    for r in range(3):
        hr = jnp.concatenate([acc_ref[r, 0], acc_ref[r, 1]], axis=1)
        t = hr * (1.0 / (deg_ref[r] + 1.0)) + b2_ref[r]
        h2 = t if h2 is None else h2 + t
    h2_ref[...] = h2
    part = jnp.concatenate(
        [jnp.sum(h2, axis=0, keepdims=True),
         jnp.sum(h2 * h2, axis=0, keepdims=True)], axis=0)

    @pl.when(i == 0)
    def _():
        st_ref[...] = part

    @pl.when(i != 0)
    def _():
        st_ref[...] = st_ref[...] + part


def _fin2_body(h2_ref, st_ref, g_ref, be_ref, out_ref):
    mu = st_ref[0:1, :] * (1.0 / N)
    var = st_ref[1:2, :] * (1.0 / N) - mu * mu
    hn = g_ref[...] * ((h2_ref[...] - mu) * lax.rsqrt(var + _EPS)) + be_ref[...]
    sg = 1.0 / (1.0 + jnp.exp(-hn))
    zmax = jnp.max(sg, axis=1, keepdims=True)
    zmin = jnp.min(sg, axis=1, keepdims=True)
    out = (sg - zmin) / (zmax - zmin)
    nrm = jnp.sqrt(jnp.sum(out * out, axis=1, keepdims=True))
    out_ref[...] = out / jnp.maximum(nrm, 1e-12)


def _sc_agg_body(width, k, with_deg, rel_stride, table_ref, i1_ref, i2_ref,
                 i3_ref, zn_ref, *rest):
    if with_deg:
        (acc_out, deg_out, acc, deg,
         srcA, srcB, dstA, dstB, rowsA, rowsB, ones_v,
         gsA, gsB, ssA, ssB) = rest
    else:
        (acc_out, acc,
         srcA, srcB, dstA, dstB, rowsA, rowsB,
         gsA, gsB, ssA, ssB) = rest
        deg = None

    c = lax.axis_index("c")
    t = lax.axis_index("s")
    row0 = t * ROWS_PT
    cN = c * N

    if with_deg:
        for j in range(SUB // 16):
            ones_v[pl.ds(j * 16, 16)] = jnp.ones((16,), jnp.float32)
        ones_v[pl.ds(SUB - 16, 16)] = jnp.ones((16,), jnp.float32)

    gmax = NBT // (2 * k)
    base = t * NBT

    def load_idx(i_ref, b0, src_v, dst_v):
        pltpu.sync_copy(i_ref.at[c, pl.ds(b0, k)], src_v)
        pltpu.sync_copy(i_ref.at[2, pl.ds(b0, k)], dst_v)

    def fire_gathers(src_v, rows_v, sem):
        for j in range(k):
            pltpu.async_copy(table_ref.at[src_v.at[j]], rows_v.at[j], sem)

    def drain_gathers(src_v, rows_v, sem):
        for j in range(k):
            pltpu.make_async_copy(table_ref.at[src_v.at[j]], rows_v.at[j],
                                  sem).wait()

    def scatter_block(acc, deg, dst_v, rows_v, sem):
        hs = []
        for j in range(k):
            hs.append(pltpu.async_copy(rows_v.at[j], acc.at[dst_v.at[j]],
                                       sem, add=True))
        if with_deg:
            @pl.when(c == 0)
            def _():
                dh = []
                for j in range(k):
                    dh.append(pltpu.async_copy(ones_v, deg.at[dst_v.at[j]],
                                               sem, add=True))
                for h in dh:
                    h.wait()
        for h in hs:
            h.wait()

    for r, i_ref in enumerate((i1_ref, i2_ref, i3_ref)):
        pltpu.sync_copy(table_ref.at[pl.ds(r * rel_stride + cN + row0,
                                           ROWS_PT)],
                        acc.at[pl.ds(row0, ROWS_PT)])
        if with_deg:
            @pl.when(c == 0)
            def _():
                pltpu.sync_copy(zn_ref.at[pl.ds(t * DEG_PT, DEG_PT)],
                                deg.at[pl.ds(t * DEG_PT, DEG_PT)])

                @pl.when(t == 0)
                def _():
                    pltpu.sync_copy(zn_ref.at[pl.ds(NS * DEG_PT, DEG_TAIL)],
                                    deg.at[pl.ds(NS * DEG_PT, DEG_TAIL)])
        plsc.subcore_barrier()

        load_idx(i_ref, base, srcA, dstA)
        fire_gathers(srcA, rowsA, gsA)

        def pipe_body(g, _, i_ref=i_ref):
            b0 = base + 2 * k * g
            load_idx(i_ref, b0 + k, srcB, dstB)
            fire_gathers(srcB, rowsB, gsB)
            drain_gathers(srcA, rowsA, gsA)
            scatter_block(acc, deg, dstA, rowsA, ssA)
            @pl.when(g < gmax - 1)
            def _():
                load_idx(i_ref, b0 + 2 * k, srcA, dstA)
                fire_gathers(srcA, rowsA, gsA)
            drain_gathers(srcB, rowsB, gsB)
            scatter_block(acc, deg, dstB, rowsB, ssB)
            return 0

        lax.fori_loop(0, gmax, pipe_body, 0)
        plsc.subcore_barrier()

        pltpu.sync_copy(acc.at[pl.ds(row0, ROWS_PT)],
                        acc_out.at[r, c, pl.ds(row0, ROWS_PT)])
        if with_deg:
            @pl.when(c == 0)
            def _():
                pltpu.sync_copy(deg.at[pl.ds(t * DEG_PT, DEG_PT)],
                                deg_out.at[r, pl.ds(t * DEG_PT, DEG_PT)])

                @pl.when(t == 0)
                def _():
                    pltpu.sync_copy(deg.at[pl.ds(NS * DEG_PT, DEG_TAIL)],
                                    deg_out.at[r, pl.ds(NS * DEG_PT,
                                                        DEG_TAIL)])
        plsc.subcore_barrier()


def _make_sc_agg(width, k, with_deg, rel_stride):
    mesh = plsc.VectorSubcoreMesh(core_axis_name="c", subcore_axis_name="s",
                                  num_cores=NC, num_subcores=NS)
    out_type = [jax.ShapeDtypeStruct((3, NC, N, width), jnp.float32)]
    scratch = [
        pltpu.VMEM_SHARED((N, width), jnp.float32),
    ]
    if with_deg:
        out_type.append(jax.ShapeDtypeStruct((3, N), jnp.float32))
        scratch.append(pltpu.VMEM_SHARED((N,), jnp.float32))
    scratch += [
        pltpu.VMEM((k, SUB), jnp.int32),
        pltpu.VMEM((k, SUB), jnp.int32),
        pltpu.VMEM((k, SUB), jnp.int32),
        pltpu.VMEM((k, SUB), jnp.int32),
        pltpu.VMEM((k, SUB, width), jnp.float32),
        pltpu.VMEM((k, SUB, width), jnp.float32),
    ]
    if with_deg:
        scratch.append(pltpu.VMEM((SUB,), jnp.float32))
    scratch += [pltpu.SemaphoreType.DMA] * 4
    return pl.kernel(
        functools.partial(_sc_agg_body, width, k, with_deg, rel_stride),
        out_type=out_type,
        mesh=mesh,
        scratch_types=scratch,
        compiler_params=pltpu.CompilerParams(use_tc_tiling_on_sc=False),
    )


def kernel(x, edge_index1, edge_index2, edge_index3,
           W1_1, b1_1, W2_1, b2_1, W3_1, b3_1,
           W1_2, b1_2, W2_2, b2_2, W3_2, b3_2,
           g1, be1, g2, be2, g3, be3):
    h_split = pl.pallas_call(
        _bn1_body,
        out_shape=jax.ShapeDtypeStruct((2 * N, 64), jnp.float32),
    )(x, g1.reshape(1, D_IN), be1.reshape(1, D_IN))

    def make_idx(e, r_bias):
        src, dst = e[0], e[1]
        return jnp.stack([src + r_bias, src + (r_bias + N), dst]
                         ).reshape(3, NBLK, SUB)

    edges = (edge_index1, edge_index2, edge_index3)
    idx1 = [make_idx(e, 0) for e in edges]
    idx2 = [make_idx(e, 2 * N * r) for r, e in enumerate(edges)]

    zeros_n = jnp.zeros((N,), jnp.float32)
    acc1, deg = _make_sc_agg(64, 4, True, 0)(
        h_split, idx1[0], idx1[1], idx1[2], zeros_n)

    deg3 = deg.reshape(3, N, 1)
    w1 = jnp.stack([W1_1, W2_1, W3_1])
    b1 = jnp.stack([b1_1, b2_1, b3_1]).reshape(3, 1, D_H)
    w2 = jnp.stack([W1_2, W2_2, W3_2])

    acc_spec = lambda w: pl.BlockSpec((3, 2, RB, w), lambda i: (0, 0, i, 0))
    deg_spec = pl.BlockSpec((3, RB, 1), lambda i: (0, i, 0))
    full = lambda *s: pl.BlockSpec(s, lambda i: (0,) * len(s))
    row_spec = lambda w: pl.BlockSpec((RB, w), lambda i: (i, 0))

    h1, st1 = pl.pallas_call(
        _mid1_body,
        grid=(GRID,),
        in_specs=[acc_spec(64), deg_spec, full(3, D_IN, D_H),
                  full(3, 1, D_H)],
        out_specs=[row_spec(D_H), full(2, D_H)],
        out_shape=[jax.ShapeDtypeStruct((N, D_H), jnp.float32),
                   jax.ShapeDtypeStruct((2, D_H), jnp.float32)],
    )(acc1, deg3, w1, b1)

    table2 = pl.pallas_call(
        _mid2_body,
        grid=(GRID,),
        in_specs=[row_spec(D_H), full(2, D_H), full(3, D_H, D_OUT),
                  full(1, D_H), full(1, D_H)],
        out_specs=pl.BlockSpec((3, 2, RB, 32), lambda i: (0, 0, i, 0)),
        out_shape=jax.ShapeDtypeStruct((3, 2, N, 32), jnp.float32),
    )(h1, st1, w2, g2.reshape(1, D_H), be2.reshape(1, D_H))

    acc2 = _make_sc_agg(32, 10, False, 2 * N)(
        table2.reshape(3 * 2 * N, 32),
        idx2[0], idx2[1], idx2[2], zeros_n)[0]

    b2 = jnp.stack([b1_2, b2_2, b3_2]).reshape(3, 1, D_OUT)
    h2, st2 = pl.pallas_call(
        _fin1_body,
        grid=(GRID,),
        in_specs=[acc_spec(32), deg_spec, full(3, 1, D_OUT)],
        out_specs=[row_spec(D_OUT), full(2, D_OUT)],
        out_shape=[jax.ShapeDtypeStruct((N, D_OUT), jnp.float32),
                   jax.ShapeDtypeStruct((2, D_OUT), jnp.float32)],
    )(acc2, deg3, b2)

    out = pl.pallas_call(
        _fin2_body,
        grid=(GRID,),
        in_specs=[row_spec(D_OUT), full(2, D_OUT), full(1, D_OUT),
                  full(1, D_OUT)],
        out_specs=row_spec(D_OUT),
        out_shape=jax.ShapeDtypeStruct((N, D_OUT), jnp.float32),
    )(h2, st2, g3.reshape(1, D_OUT), be3.reshape(1, D_OUT))
    return out


# speedup vs baseline: 9.5664x; 1.0848x over previous
import functools

import jax
import jax.numpy as jnp
from jax import lax
from jax.experimental import pallas as pl
from jax.experimental.pallas import tpu as pltpu
from jax.experimental.pallas import tpu_sc as plsc

N = 10000
E = 320000
D_IN = 128
D_H = 128
D_OUT = 64

NC = 2
NS = 16
SUB = 80
NBLK = E // SUB
NBT = NBLK // NS
ROWS_PT = N // NS
DEG_PT = 624
DEG_TAIL = N - NS * DEG_PT

_EPS = 1e-5


def _bn1_body(x_ref, g_ref, be_ref, out_ref):
    x = x_ref[...]
    mu = jnp.mean(x, axis=0, keepdims=True)
    xc = x - mu
    var = jnp.mean(xc * xc, axis=0, keepdims=True)
    hn = g_ref[...] * (xc * lax.rsqrt(var + _EPS)) + be_ref[...]
    out_ref[0:N, :] = hn[:, 0:64]
    out_ref[N:2 * N, :] = hn[:, 64:128]


RB = 1000
GRID = N // RB


def _mid1_body(acc_ref, deg_ref, w1_ref, b1_ref, h1_ref, st_ref):
    i = pl.program_id(0)
    h1 = None
    for r in range(3):
        hr = jnp.concatenate([acc_ref[r, 0], acc_ref[r, 1]], axis=1)
        hr = hr * (1.0 / (deg_ref[r] + 1.0))
        t = jnp.dot(hr, w1_ref[r], preferred_element_type=jnp.float32)
        t = t + b1_ref[r]
        h1 = t if h1 is None else h1 + t
    h1 = jnp.maximum(h1, 0.0)
    h1_ref[...] = h1
    part = jnp.concatenate(
        [jnp.sum(h1, axis=0, keepdims=True),
         jnp.sum(h1 * h1, axis=0, keepdims=True)], axis=0)

    @pl.when(i == 0)
    def _():
        st_ref[...] = part

    @pl.when(i != 0)
    def _():
        st_ref[...] = st_ref[...] + part


def _mid2_body(h1_ref, st_ref, w2_ref, g_ref, be_ref, out_ref):
    mu = st_ref[0:1, :] * (1.0 / N)
    var = st_ref[1:2, :] * (1.0 / N) - mu * mu
    hb = g_ref[...] * ((h1_ref[...] - mu) * lax.rsqrt(var + _EPS)) + be_ref[...]
    for r in range(3):
        y = jnp.dot(hb, w2_ref[r], preferred_element_type=jnp.float32)
        out_ref[r, 0] = y[:, 0:32]
        out_ref[r, 1] = y[:, 32:64]


def _fin1_body(acc_ref, deg_ref, b2_ref, h2_ref, st_ref):
    i = pl.program_id(0)
    h2 = None
    for r in range(3):
        hr = jnp.concatenate([acc_ref[r, 0], acc_ref[r, 1]], axis=1)
        t = hr * (1.0 / (deg_ref[r] + 1.0)) + b2_ref[r]
        h2 = t if h2 is None else h2 + t
    h2_ref[...] = h2
    part = jnp.concatenate(
        [jnp.sum(h2, axis=0, keepdims=True),
         jnp.sum(h2 * h2, axis=0, keepdims=True)], axis=0)

    @pl.when(i == 0)
    def _():
        st_ref[...] = part

    @pl.when(i != 0)
    def _():
        st_ref[...] = st_ref[...] + part


def _fin2_body(h2_ref, st_ref, g_ref, be_ref, out_ref):
    mu = st_ref[0:1, :] * (1.0 / N)
    var = st_ref[1:2, :] * (1.0 / N) - mu * mu
    hn = g_ref[...] * ((h2_ref[...] - mu) * lax.rsqrt(var + _EPS)) + be_ref[...]
    sg = 1.0 / (1.0 + jnp.exp(-hn))
    zmax = jnp.max(sg, axis=1, keepdims=True)
    zmin = jnp.min(sg, axis=1, keepdims=True)
    out = (sg - zmin) / (zmax - zmin)
    nrm = jnp.sqrt(jnp.sum(out * out, axis=1, keepdims=True))
    out_ref[...] = out / jnp.maximum(nrm, 1e-12)


def _sc_agg_body(width, k, with_deg, rel_stride, table_ref, e1_ref, e2_ref,
                 e3_ref, *rest):
    if with_deg:
        (acc_out, deg_out, acc, deg,
         srcA, srcB, dstA, dstB, rowsA, rowsB, ones_v, zer_v,
         gsA, gsB, ssA, ssB) = rest
    else:
        (acc_out, acc,
         srcA, srcB, dstA, dstB, rowsA, rowsB,
         gsA, gsB, ssA, ssB) = rest
        deg = None

    c = lax.axis_index("c")
    t = lax.axis_index("s")
    row0 = t * ROWS_PT
    cN = c * N

    if with_deg:
        for j in range(SUB // 16):
            ones_v[pl.ds(j * 16, 16)] = jnp.ones((16,), jnp.float32)
        for j in range(DEG_PT // 16):
            zer_v[pl.ds(j * 16, 16)] = jnp.zeros((16,), jnp.float32)

    gmax = NBT // (2 * k)
    base = t * NBT

    def load_idx(e_ref, b0, src_v, dst_v, bias):
        pltpu.sync_copy(e_ref.at[0, pl.ds(b0, k)], src_v)
        pltpu.sync_copy(e_ref.at[1, pl.ds(b0, k)], dst_v)
        for j in range(k):
            for m in range(SUB // 16):
                sl = pl.ds(m * 16, 16)
                src_v[j, sl] = src_v[j, sl] + bias

    def fire_gathers(src_v, rows_v, sem):
        for j in range(k):
            pltpu.async_copy(table_ref.at[src_v.at[j]], rows_v.at[j], sem)

    def drain_gathers(src_v, rows_v, sem):
        for j in range(k):
            pltpu.make_async_copy(table_ref.at[src_v.at[j]], rows_v.at[j],
                                  sem).wait()

    def scatter_block(acc, deg, dst_v, rows_v, sem):
        hs = []
        for j in range(k):
            hs.append(pltpu.async_copy(rows_v.at[j], acc.at[dst_v.at[j]],
                                       sem, add=True))
        if with_deg:
            @pl.when(c == 0)
            def _():
                dh = []
                for j in range(k):
                    dh.append(pltpu.async_copy(ones_v, deg.at[dst_v.at[j]],
                                               sem, add=True))
                for h in dh:
                    h.wait()
        for h in hs:
            h.wait()

    for r, e_ref in enumerate((e1_ref, e2_ref, e3_ref)):
        bias = r * rel_stride + cN
        pltpu.sync_copy(table_ref.at[pl.ds(r * rel_stride + cN + row0,
                                           ROWS_PT)],
                        acc.at[pl.ds(row0, ROWS_PT)])
        if with_deg:
            @pl.when(c == 0)
            def _():
                pltpu.sync_copy(zer_v, deg.at[pl.ds(t * DEG_PT, DEG_PT)])

                @pl.when(t == 0)
                def _():
                    pltpu.sync_copy(zer_v.at[pl.ds(0, DEG_TAIL)],
                                    deg.at[pl.ds(NS * DEG_PT, DEG_TAIL)])
        plsc.subcore_barrier()

        load_idx(e_ref, base, srcA, dstA, bias)
        fire_gathers(srcA, rowsA, gsA)

        def pipe_body(g, _, e_ref=e_ref, bias=bias):
            b0 = base + 2 * k * g
            load_idx(e_ref, b0 + k, srcB, dstB, bias)
            fire_gathers(srcB, rowsB, gsB)
            drain_gathers(srcA, rowsA, gsA)
            scatter_block(acc, deg, dstA, rowsA, ssA)
            @pl.when(g < gmax - 1)
            def _():
                load_idx(e_ref, b0 + 2 * k, srcA, dstA, bias)
                fire_gathers(srcA, rowsA, gsA)
            drain_gathers(srcB, rowsB, gsB)
            scatter_block(acc, deg, dstB, rowsB, ssB)
            return 0

        lax.fori_loop(0, gmax, pipe_body, 0)
        plsc.subcore_barrier()

        pltpu.sync_copy(acc.at[pl.ds(row0, ROWS_PT)],
                        acc_out.at[r, c, pl.ds(row0, ROWS_PT)])
        if with_deg:
            @pl.when(c == 0)
            def _():
                pltpu.sync_copy(deg.at[pl.ds(t * DEG_PT, DEG_PT)],
                                deg_out.at[r, pl.ds(t * DEG_PT, DEG_PT)])

                @pl.when(t == 0)
                def _():
                    pltpu.sync_copy(deg.at[pl.ds(NS * DEG_PT, DEG_TAIL)],
                                    deg_out.at[r, pl.ds(NS * DEG_PT,
                                                        DEG_TAIL)])
        plsc.subcore_barrier()


def _make_sc_agg(width, k, with_deg, rel_stride):
    mesh = plsc.VectorSubcoreMesh(core_axis_name="c", subcore_axis_name="s",
                                  num_cores=NC, num_subcores=NS)
    out_type = [jax.ShapeDtypeStruct((3, NC, N, width), jnp.float32)]
    scratch = [
        pltpu.VMEM_SHARED((N, width), jnp.float32),
    ]
    if with_deg:
        out_type.append(jax.ShapeDtypeStruct((3, N), jnp.float32))
        scratch.append(pltpu.VMEM_SHARED((N,), jnp.float32))
    scratch += [
        pltpu.VMEM((k, SUB), jnp.int32),
        pltpu.VMEM((k, SUB), jnp.int32),
        pltpu.VMEM((k, SUB), jnp.int32),
        pltpu.VMEM((k, SUB), jnp.int32),
        pltpu.VMEM((k, SUB, width), jnp.float32),
        pltpu.VMEM((k, SUB, width), jnp.float32),
    ]
    if with_deg:
        scratch.append(pltpu.VMEM((SUB,), jnp.float32))
        scratch.append(pltpu.VMEM((DEG_PT,), jnp.float32))
    scratch += [pltpu.SemaphoreType.DMA] * 4
    return pl.kernel(
        functools.partial(_sc_agg_body, width, k, with_deg, rel_stride),
        out_type=out_type,
        mesh=mesh,
        scratch_types=scratch,
        compiler_params=pltpu.CompilerParams(use_tc_tiling_on_sc=False),
    )


def kernel(x, edge_index1, edge_index2, edge_index3,
           W1_1, b1_1, W2_1, b2_1, W3_1, b3_1,
           W1_2, b1_2, W2_2, b2_2, W3_2, b3_2,
           g1, be1, g2, be2, g3, be3):
    h_split = pl.pallas_call(
        _bn1_body,
        out_shape=jax.ShapeDtypeStruct((2 * N, 64), jnp.float32),
    )(x, g1.reshape(1, D_IN), be1.reshape(1, D_IN))

    e1 = edge_index1.reshape(2, NBLK, SUB)
    e2 = edge_index2.reshape(2, NBLK, SUB)
    e3 = edge_index3.reshape(2, NBLK, SUB)

    acc1, deg = _make_sc_agg(64, 5, True, 0)(h_split, e1, e2, e3)

    deg3 = deg.reshape(3, N, 1)
    w1 = jnp.stack([W1_1, W2_1, W3_1])
    b1 = jnp.stack([b1_1, b2_1, b3_1]).reshape(3, 1, D_H)
    w2 = jnp.stack([W1_2, W2_2, W3_2])

    acc_spec = lambda w: pl.BlockSpec((3, 2, RB, w), lambda i: (0, 0, i, 0))
    deg_spec = pl.BlockSpec((3, RB, 1), lambda i: (0, i, 0))
    full = lambda *s: pl.BlockSpec(s, lambda i: (0,) * len(s))
    row_spec = lambda w: pl.BlockSpec((RB, w), lambda i: (i, 0))

    h1, st1 = pl.pallas_call(
        _mid1_body,
        grid=(GRID,),
        in_specs=[acc_spec(64), deg_spec, full(3, D_IN, D_H),
                  full(3, 1, D_H)],
        out_specs=[row_spec(D_H), full(2, D_H)],
        out_shape=[jax.ShapeDtypeStruct((N, D_H), jnp.float32),
                   jax.ShapeDtypeStruct((2, D_H), jnp.float32)],
    )(acc1, deg3, w1, b1)

    table2 = pl.pallas_call(
        _mid2_body,
        grid=(GRID,),
        in_specs=[row_spec(D_H), full(2, D_H), full(3, D_H, D_OUT),
                  full(1, D_H), full(1, D_H)],
        out_specs=pl.BlockSpec((3, 2, RB, 32), lambda i: (0, 0, i, 0)),
        out_shape=jax.ShapeDtypeStruct((3, 2, N, 32), jnp.float32),
    )(h1, st1, w2, g2.reshape(1, D_H), be2.reshape(1, D_H))

    acc2 = _make_sc_agg(32, 5, False, 2 * N)(
        table2.reshape(3 * 2 * N, 32), e1, e2, e3)[0]

    b2 = jnp.stack([b1_2, b2_2, b3_2]).reshape(3, 1, D_OUT)
    h2, st2 = pl.pallas_call(
        _fin1_body,
        grid=(GRID,),
        in_specs=[acc_spec(32), deg_spec, full(3, 1, D_OUT)],
        out_specs=[row_spec(D_OUT), full(2, D_OUT)],
        out_shape=[jax.ShapeDtypeStruct((N, D_OUT), jnp.float32),
                   jax.ShapeDtypeStruct((2, D_OUT), jnp.float32)],
    )(acc2, deg3, b2)

    out = pl.pallas_call(
        _fin2_body,
        grid=(GRID,),
        in_specs=[row_spec(D_OUT), full(2, D_OUT), full(1, D_OUT),
                  full(1, D_OUT)],
        out_specs=row_spec(D_OUT),
        out_shape=jax.ShapeDtypeStruct((N, D_OUT), jnp.float32),
    )(h2, st2, g3.reshape(1, D_OUT), be3.reshape(1, D_OUT))
    return out


# speedup vs baseline: 11.1515x; 1.1657x over previous
import functools

import jax
import jax.numpy as jnp
from jax import lax
from jax.experimental import pallas as pl
from jax.experimental.pallas import tpu as pltpu
from jax.experimental.pallas import tpu_sc as plsc

N = 10000
E = 320000
D_IN = 128
D_H = 128
D_OUT = 64

NC = 2
NS = 16
SUB = 80
NBLK = E // SUB
NBT = NBLK // NS
ROWS_PT = N // NS
DEG_PT = 624
DEG_TAIL = N - NS * DEG_PT

_EPS = 1e-5


def _bn1_body(x_ref, g_ref, be_ref, out_ref):
    x = x_ref[...]
    mu = jnp.mean(x, axis=0, keepdims=True)
    xc = x - mu
    var = jnp.mean(xc * xc, axis=0, keepdims=True)
    hn = g_ref[...] * (xc * lax.rsqrt(var + _EPS)) + be_ref[...]
    out_ref[0:N, :] = hn[:, 0:64]
    out_ref[N:2 * N, :] = hn[:, 64:128]


RB = 1000
GRID = N // RB


def _mid1_body(acc_ref, deg_ref, w1_ref, b1_ref, h1_ref, st_ref):
    i = pl.program_id(0)
    h1 = None
    for r in range(3):
        hr = jnp.concatenate([acc_ref[r, 0], acc_ref[r, 1]], axis=1)
        hr = hr * (1.0 / (deg_ref[r] + 1.0))
        t = jnp.dot(hr, w1_ref[r], preferred_element_type=jnp.float32)
        t = t + b1_ref[r]
        h1 = t if h1 is None else h1 + t
    h1 = jnp.maximum(h1, 0.0)
    h1_ref[...] = h1
    part = jnp.concatenate(
        [jnp.sum(h1, axis=0, keepdims=True),
         jnp.sum(h1 * h1, axis=0, keepdims=True)], axis=0)

    @pl.when(i == 0)
    def _():
        st_ref[...] = part

    @pl.when(i != 0)
    def _():
        st_ref[...] = st_ref[...] + part


def _mid2_body(h1_ref, st_ref, w2_ref, g_ref, be_ref, out_ref):
    mu = st_ref[0:1, :] * (1.0 / N)
    var = st_ref[1:2, :] * (1.0 / N) - mu * mu
    hb = g_ref[...] * ((h1_ref[...] - mu) * lax.rsqrt(var + _EPS)) + be_ref[...]
    for r in range(3):
        y = jnp.dot(hb, w2_ref[r], preferred_element_type=jnp.float32)
        out_ref[r, 0] = y[:, 0:32]
        out_ref[r, 1] = y[:, 32:64]


def _fin1_body(acc_ref, deg_ref, b2_ref, h2_ref, st_ref):
    i = pl.program_id(0)
    h2 = None
    for r in range(3):
        hr = jnp.concatenate([acc_ref[r, 0], acc_ref[r, 1]], axis=1)
        t = hr * (1.0 / (deg_ref[r] + 1.0)) + b2_ref[r]
        h2 = t if h2 is None else h2 + t
    h2_ref[...] = h2
    part = jnp.concatenate(
        [jnp.sum(h2, axis=0, keepdims=True),
         jnp.sum(h2 * h2, axis=0, keepdims=True)], axis=0)

    @pl.when(i == 0)
    def _():
        st_ref[...] = part

    @pl.when(i != 0)
    def _():
        st_ref[...] = st_ref[...] + part


def _fin2_body(h2_ref, st_ref, g_ref, be_ref, out_ref):
    mu = st_ref[0:1, :] * (1.0 / N)
    var = st_ref[1:2, :] * (1.0 / N) - mu * mu
    hn = g_ref[...] * ((h2_ref[...] - mu) * lax.rsqrt(var + _EPS)) + be_ref[...]
    sg = 1.0 / (1.0 + jnp.exp(-hn))
    zmax = jnp.max(sg, axis=1, keepdims=True)
    zmin = jnp.min(sg, axis=1, keepdims=True)
    out = (sg - zmin) / (zmax - zmin)
    nrm = jnp.sqrt(jnp.sum(out * out, axis=1, keepdims=True))
    out_ref[...] = out / jnp.maximum(nrm, 1e-12)


def _sc_agg_body(width, k, with_deg, rel_stride, table_ref, e1_ref, e2_ref,
                 e3_ref, *rest):
    if with_deg:
        (acc_out, deg_out, acc, deg,
         src0, src1, src2, dst0, dst1, dst2, rows0, rows1, rows2,
         ones_v, zer_v, gs0, gs1, gs2, ss0, ss1, ss2) = rest
    else:
        (acc_out, acc,
         src0, src1, src2, dst0, dst1, dst2, rows0, rows1, rows2,
         gs0, gs1, gs2, ss0, ss1, ss2) = rest
        deg = None
    srcs = (src0, src1, src2)
    dsts = (dst0, dst1, dst2)
    rows = (rows0, rows1, rows2)
    gss = (gs0, gs1, gs2)
    sss = (ss0, ss1, ss2)

    c = lax.axis_index("c")
    t = lax.axis_index("s")
    row0 = t * ROWS_PT
    cN = c * N

    if with_deg:
        for j in range(SUB // 16):
            ones_v[pl.ds(j * 16, 16)] = jnp.ones((16,), jnp.float32)
        for j in range(DEG_PT // 16):
            zer_v[pl.ds(j * 16, 16)] = jnp.zeros((16,), jnp.float32)

    nphase = NBT // k
    ngrp = (nphase - 2) // 3
    base = t * NBT

    def load_idx(e_ref, b0, src_v, dst_v, bias):
        pltpu.sync_copy(e_ref.at[0, pl.ds(b0, k)], src_v)
        pltpu.sync_copy(e_ref.at[1, pl.ds(b0, k)], dst_v)
        for j in range(k):
            for m in range(SUB // 16):
                sl = pl.ds(m * 16, 16)
                src_v[j, sl] = src_v[j, sl] + bias

    def phase_fire(e_ref, bias, b, blk):
        load_idx(e_ref, blk, srcs[b], dsts[b], bias)
        for j in range(k):
            pltpu.async_copy(table_ref.at[srcs[b].at[j]], rows[b].at[j],
                             gss[b])

    def phase_complete(acc, deg, b):
        for j in range(k):
            pltpu.make_async_copy(table_ref.at[srcs[b].at[j]],
                                  rows[b].at[j], gss[b]).wait()
        for j in range(k):
            pltpu.async_copy(rows[b].at[j], acc.at[dsts[b].at[j]],
                             sss[b], add=True)
        if with_deg:
            @pl.when(c == 0)
            def _():
                for j in range(k):
                    pltpu.async_copy(ones_v, deg.at[dsts[b].at[j]],
                                     sss[b], add=True)

    def drain_scatters(acc, deg, b):
        for j in range(k):
            pltpu.make_async_copy(rows[b].at[j], acc.at[dsts[b].at[j]],
                                  sss[b]).wait()
        if with_deg:
            @pl.when(c == 0)
            def _():
                for j in range(k):
                    pltpu.make_async_copy(ones_v, deg.at[dsts[b].at[j]],
                                          sss[b]).wait()

    for r, e_ref in enumerate((e1_ref, e2_ref, e3_ref)):
        bias = r * rel_stride + cN
        pltpu.sync_copy(table_ref.at[pl.ds(r * rel_stride + cN + row0,
                                           ROWS_PT)],
                        acc.at[pl.ds(row0, ROWS_PT)])
        if with_deg:
            @pl.when(c == 0)
            def _():
                pltpu.sync_copy(zer_v, deg.at[pl.ds(t * DEG_PT, DEG_PT)])

                @pl.when(t == 0)
                def _():
                    pltpu.sync_copy(zer_v.at[pl.ds(0, DEG_TAIL)],
                                    deg.at[pl.ds(NS * DEG_PT, DEG_TAIL)])
        plsc.subcore_barrier()

        phase_fire(e_ref, bias, 0, base)
        phase_fire(e_ref, bias, 1, base + k)
        phase_complete(acc, deg, 0)

        def ring_body(g, _, e_ref=e_ref, bias=bias):
            b0 = base + (2 + 3 * g) * k
            @pl.when(g >= 1)
            def _():
                drain_scatters(acc, deg, 2)
            phase_fire(e_ref, bias, 2, b0)
            phase_complete(acc, deg, 1)
            drain_scatters(acc, deg, 0)
            phase_fire(e_ref, bias, 0, b0 + k)
            phase_complete(acc, deg, 2)
            drain_scatters(acc, deg, 1)
            phase_fire(e_ref, bias, 1, b0 + 2 * k)
            phase_complete(acc, deg, 0)
            return 0

        lax.fori_loop(0, ngrp, ring_body, 0)
        phase_complete(acc, deg, 1)
        drain_scatters(acc, deg, 2)
        drain_scatters(acc, deg, 0)
        drain_scatters(acc, deg, 1)
        plsc.subcore_barrier()

        pltpu.sync_copy(acc.at[pl.ds(row0, ROWS_PT)],
                        acc_out.at[r, c, pl.ds(row0, ROWS_PT)])
        if with_deg:
            @pl.when(c == 0)
            def _():
                pltpu.sync_copy(deg.at[pl.ds(t * DEG_PT, DEG_PT)],
                                deg_out.at[r, pl.ds(t * DEG_PT, DEG_PT)])

                @pl.when(t == 0)
                def _():
                    pltpu.sync_copy(deg.at[pl.ds(NS * DEG_PT, DEG_TAIL)],
                                    deg_out.at[r, pl.ds(NS * DEG_PT,
                                                        DEG_TAIL)])
        plsc.subcore_barrier()


def _make_sc_agg(width, k, with_deg, rel_stride):
    mesh = plsc.VectorSubcoreMesh(core_axis_name="c", subcore_axis_name="s",
                                  num_cores=NC, num_subcores=NS)
    out_type = [jax.ShapeDtypeStruct((3, NC, N, width), jnp.float32)]
    scratch = [
        pltpu.VMEM_SHARED((N, width), jnp.float32),
    ]
    if with_deg:
        out_type.append(jax.ShapeDtypeStruct((3, N), jnp.float32))
        scratch.append(pltpu.VMEM_SHARED((N,), jnp.float32))
    scratch += [pltpu.VMEM((k, SUB), jnp.int32)] * 6
    scratch += [pltpu.VMEM((k, SUB, width), jnp.float32)] * 3
    if with_deg:
        scratch.append(pltpu.VMEM((SUB,), jnp.float32))
        scratch.append(pltpu.VMEM((DEG_PT,), jnp.float32))
    scratch += [pltpu.SemaphoreType.DMA] * 6
    return pl.kernel(
        functools.partial(_sc_agg_body, width, k, with_deg, rel_stride),
        out_type=out_type,
        mesh=mesh,
        scratch_types=scratch,
        compiler_params=pltpu.CompilerParams(use_tc_tiling_on_sc=False),
    )


def kernel(x, edge_index1, edge_index2, edge_index3,
           W1_1, b1_1, W2_1, b2_1, W3_1, b3_1,
           W1_2, b1_2, W2_2, b2_2, W3_2, b3_2,
           g1, be1, g2, be2, g3, be3):
    h_split = pl.pallas_call(
        _bn1_body,
        out_shape=jax.ShapeDtypeStruct((2 * N, 64), jnp.float32),
    )(x, g1.reshape(1, D_IN), be1.reshape(1, D_IN))

    e1 = edge_index1.reshape(2, NBLK, SUB)
    e2 = edge_index2.reshape(2, NBLK, SUB)
    e3 = edge_index3.reshape(2, NBLK, SUB)

    acc1, deg = _make_sc_agg(64, 5, True, 0)(h_split, e1, e2, e3)

    deg3 = deg.reshape(3, N, 1)
    w1 = jnp.stack([W1_1, W2_1, W3_1])
    b1 = jnp.stack([b1_1, b2_1, b3_1]).reshape(3, 1, D_H)
    w2 = jnp.stack([W1_2, W2_2, W3_2])

    acc_spec = lambda w: pl.BlockSpec((3, 2, RB, w), lambda i: (0, 0, i, 0))
    deg_spec = pl.BlockSpec((3, RB, 1), lambda i: (0, i, 0))
    full = lambda *s: pl.BlockSpec(s, lambda i: (0,) * len(s))
    row_spec = lambda w: pl.BlockSpec((RB, w), lambda i: (i, 0))

    h1, st1 = pl.pallas_call(
        _mid1_body,
        grid=(GRID,),
        in_specs=[acc_spec(64), deg_spec, full(3, D_IN, D_H),
                  full(3, 1, D_H)],
        out_specs=[row_spec(D_H), full(2, D_H)],
        out_shape=[jax.ShapeDtypeStruct((N, D_H), jnp.float32),
                   jax.ShapeDtypeStruct((2, D_H), jnp.float32)],
    )(acc1, deg3, w1, b1)

    table2 = pl.pallas_call(
        _mid2_body,
        grid=(GRID,),
        in_specs=[row_spec(D_H), full(2, D_H), full(3, D_H, D_OUT),
                  full(1, D_H), full(1, D_H)],
        out_specs=pl.BlockSpec((3, 2, RB, 32), lambda i: (0, 0, i, 0)),
        out_shape=jax.ShapeDtypeStruct((3, 2, N, 32), jnp.float32),
    )(h1, st1, w2, g2.reshape(1, D_H), be2.reshape(1, D_H))

    acc2 = _make_sc_agg(32, 5, False, 2 * N)(
        table2.reshape(3 * 2 * N, 32), e1, e2, e3)[0]

    b2 = jnp.stack([b1_2, b2_2, b3_2]).reshape(3, 1, D_OUT)
    h2, st2 = pl.pallas_call(
        _fin1_body,
        grid=(GRID,),
        in_specs=[acc_spec(32), deg_spec, full(3, 1, D_OUT)],
        out_specs=[row_spec(D_OUT), full(2, D_OUT)],
        out_shape=[jax.ShapeDtypeStruct((N, D_OUT), jnp.float32),
                   jax.ShapeDtypeStruct((2, D_OUT), jnp.float32)],
    )(acc2, deg3, b2)

    out = pl.pallas_call(
        _fin2_body,
        grid=(GRID,),
        in_specs=[row_spec(D_OUT), full(2, D_OUT), full(1, D_OUT),
                  full(1, D_OUT)],
        out_specs=row_spec(D_OUT),
        out_shape=jax.ShapeDtypeStruct((N, D_OUT), jnp.float32),
    )(h2, st2, g3.reshape(1, D_OUT), be3.reshape(1, D_OUT))
    return out


# speedup vs baseline: 12.6757x; 1.1367x over previous
import functools

import jax
import jax.numpy as jnp
from jax import lax
from jax.experimental import pallas as pl
from jax.experimental.pallas import tpu as pltpu
from jax.experimental.pallas import tpu_sc as plsc

N = 10000
E = 320000
D_IN = 128
D_H = 128
D_OUT = 64

NC = 2
NS = 16
SUB = 80
NBLK = E // SUB
NBT = NBLK // NS
ROWS_PT = N // NS
DEG_PT = 624
DEG_TAIL = N - NS * DEG_PT

_EPS = 1e-5


def _bn1_body(x_ref, g_ref, be_ref, out_ref):
    x = x_ref[...]
    mu = jnp.mean(x, axis=0, keepdims=True)
    xc = x - mu
    var = jnp.mean(xc * xc, axis=0, keepdims=True)
    hn = g_ref[...] * (xc * lax.rsqrt(var + _EPS)) + be_ref[...]
    out_ref[0:N, :] = hn[:, 0:64]
    out_ref[N:2 * N, :] = hn[:, 64:128]


RB = 1000
GRID = N // RB


def _mid1_body(acc_ref, deg_ref, w1_ref, b1_ref, h1_ref, st_ref):
    i = pl.program_id(0)
    h1 = None
    for r in range(3):
        hr = jnp.concatenate([acc_ref[r, 0], acc_ref[r, 1]], axis=1)
        hr = hr * (1.0 / (deg_ref[r] + 1.0))
        t = jnp.dot(hr, w1_ref[r], preferred_element_type=jnp.float32)
        t = t + b1_ref[r]
        h1 = t if h1 is None else h1 + t
    h1 = jnp.maximum(h1, 0.0)
    h1_ref[...] = h1
    part = jnp.concatenate(
        [jnp.sum(h1, axis=0, keepdims=True),
         jnp.sum(h1 * h1, axis=0, keepdims=True)], axis=0)

    @pl.when(i == 0)
    def _():
        st_ref[...] = part

    @pl.when(i != 0)
    def _():
        st_ref[...] = st_ref[...] + part


def _mid2_body(h1_ref, st_ref, w2_ref, g_ref, be_ref, out_ref):
    mu = st_ref[0:1, :] * (1.0 / N)
    var = st_ref[1:2, :] * (1.0 / N) - mu * mu
    hb = g_ref[...] * ((h1_ref[...] - mu) * lax.rsqrt(var + _EPS)) + be_ref[...]
    for r in range(3):
        y = jnp.dot(hb, w2_ref[r], preferred_element_type=jnp.float32)
        out_ref[r, 0] = y[:, 0:32]
        out_ref[r, 1] = y[:, 32:64]


def _fin1_body(acc_ref, deg_ref, b2_ref, h2_ref, st_ref):
    i = pl.program_id(0)
    h2 = None
    for r in range(3):
        hr = jnp.concatenate([acc_ref[r, 0], acc_ref[r, 1]], axis=1)
        t = hr * (1.0 / (deg_ref[r] + 1.0)) + b2_ref[r]
        h2 = t if h2 is None else h2 + t
    h2_ref[...] = h2
    part = jnp.concatenate(
        [jnp.sum(h2, axis=0, keepdims=True),
         jnp.sum(h2 * h2, axis=0, keepdims=True)], axis=0)

    @pl.when(i == 0)
    def _():
        st_ref[...] = part

    @pl.when(i != 0)
    def _():
        st_ref[...] = st_ref[...] + part


def _fin2_body(h2_ref, st_ref, g_ref, be_ref, out_ref):
    mu = st_ref[0:1, :] * (1.0 / N)
    var = st_ref[1:2, :] * (1.0 / N) - mu * mu
    hn = g_ref[...] * ((h2_ref[...] - mu) * lax.rsqrt(var + _EPS)) + be_ref[...]
    sg = 1.0 / (1.0 + jnp.exp(-hn))
    zmax = jnp.max(sg, axis=1, keepdims=True)
    zmin = jnp.min(sg, axis=1, keepdims=True)
    out = (sg - zmin) / (zmax - zmin)
    nrm = jnp.sqrt(jnp.sum(out * out, axis=1, keepdims=True))
    out_ref[...] = out / jnp.maximum(nrm, 1e-12)


def _sc_agg_body(width, k, with_deg, rel_stride, table_ref, e1_ref, e2_ref,
                 e3_ref, *rest):
    if with_deg:
        (acc_out, deg_out, acc, deg,
         src0, src1, src2, dst0, dst1, dst2, rows0, rows1, rows2,
         ones_v, zer_v, gs0, gs1, gs2, ss0, ss1, ss2,
         is0, is1, is2, id0, id1, id2) = rest
    else:
        (acc_out, acc,
         src0, src1, src2, dst0, dst1, dst2, rows0, rows1, rows2,
         gs0, gs1, gs2, ss0, ss1, ss2,
         is0, is1, is2, id0, id1, id2) = rest
        deg = None
    srcs = (src0, src1, src2)
    dsts = (dst0, dst1, dst2)
    rows = (rows0, rows1, rows2)
    gss = (gs0, gs1, gs2)
    sss = (ss0, ss1, ss2)
    iss = (is0, is1, is2)
    ids = (id0, id1, id2)

    c = lax.axis_index("c")
    t = lax.axis_index("s")
    row0 = t * ROWS_PT
    cN = c * N

    if with_deg:
        for j in range(k):
            for m in range(SUB // 16):
                ones_v[j, pl.ds(m * 16, 16)] = jnp.ones((16,), jnp.float32)
        for j in range(DEG_PT // 16):
            zer_v[pl.ds(j * 16, 16)] = jnp.zeros((16,), jnp.float32)

    nphase = NBT // k
    ngrp = (nphase - 2) // 3
    base = t * NBT

    def fire_src(e_ref, b, blk):
        pltpu.async_copy(e_ref.at[0, pl.ds(blk, k)], srcs[b], iss[b])

    def phase_fire(e_ref, bias, b, blk):
        pltpu.make_async_copy(e_ref.at[0, pl.ds(blk, k)], srcs[b],
                              iss[b]).wait()
        for j in range(k):
            for m in range(SUB // 16):
                sl = pl.ds(m * 16, 16)
                srcs[b][j, sl] = srcs[b][j, sl] + bias
        pltpu.async_copy(e_ref.at[1, pl.ds(blk, k)], dsts[b], ids[b])
        for j in range(k):
            pltpu.async_copy(table_ref.at[srcs[b].at[j]], rows[b].at[j],
                             gss[b])

    def phase_complete(e_ref, acc, deg, b, blk):
        for j in range(k):
            pltpu.make_async_copy(table_ref.at[srcs[b].at[j]],
                                  rows[b].at[j], gss[b]).wait()
        pltpu.make_async_copy(e_ref.at[1, pl.ds(blk, k)], dsts[b],
                              ids[b]).wait()
        for j in range(k):
            pltpu.async_copy(rows[b].at[j], acc.at[dsts[b].at[j]],
                             sss[b], add=True)
        if with_deg:
            @pl.when(c == 0)
            def _():
                for j in range(k):
                    pltpu.async_copy(ones_v.at[j], deg.at[dsts[b].at[j]],
                                     sss[b], add=True)

    def drain_scatters(acc, deg, b):
        for j in range(k):
            pltpu.make_async_copy(rows[b].at[j], acc.at[dsts[b].at[j]],
                                  sss[b]).wait()
        if with_deg:
            @pl.when(c == 0)
            def _():
                for j in range(k):
                    pltpu.make_async_copy(ones_v.at[j], deg.at[dsts[b].at[j]],
                                          sss[b]).wait()

    for r, e_ref in enumerate((e1_ref, e2_ref, e3_ref)):
        bias = r * rel_stride + cN
        pltpu.sync_copy(table_ref.at[pl.ds(r * rel_stride + cN + row0,
                                           ROWS_PT)],
                        acc.at[pl.ds(row0, ROWS_PT)])
        if with_deg:
            @pl.when(c == 0)
            def _():
                pltpu.sync_copy(zer_v, deg.at[pl.ds(t * DEG_PT, DEG_PT)])

                @pl.when(t == 0)
                def _():
                    pltpu.sync_copy(zer_v.at[pl.ds(0, DEG_TAIL)],
                                    deg.at[pl.ds(NS * DEG_PT, DEG_TAIL)])
        plsc.subcore_barrier()

        fire_src(e_ref, 0, base)
        fire_src(e_ref, 1, base + k)
        phase_fire(e_ref, bias, 0, base)
        fire_src(e_ref, 2, base + 2 * k)
        phase_fire(e_ref, bias, 1, base + k)
        phase_complete(e_ref, acc, deg, 0, base)

        def ring_body(g, _, e_ref=e_ref, bias=bias):
            b0 = base + (2 + 3 * g) * k
            @pl.when(g >= 1)
            def _():
                drain_scatters(acc, deg, 2)
            phase_fire(e_ref, bias, 2, b0)
            fire_src(e_ref, 0, b0 + k)
            phase_complete(e_ref, acc, deg, 1, b0 - k)
            drain_scatters(acc, deg, 0)
            phase_fire(e_ref, bias, 0, b0 + k)
            fire_src(e_ref, 1, b0 + 2 * k)
            phase_complete(e_ref, acc, deg, 2, b0)
            drain_scatters(acc, deg, 1)
            phase_fire(e_ref, bias, 1, b0 + 2 * k)

            @pl.when(g < ngrp - 1)
            def _():
                fire_src(e_ref, 2, b0 + 3 * k)
            phase_complete(e_ref, acc, deg, 0, b0 + k)
            return 0

        lax.fori_loop(0, ngrp, ring_body, 0)
        phase_complete(e_ref, acc, deg, 1, base + (nphase - 1) * k)
        drain_scatters(acc, deg, 2)
        drain_scatters(acc, deg, 0)
        drain_scatters(acc, deg, 1)
        plsc.subcore_barrier()

        pltpu.sync_copy(acc.at[pl.ds(row0, ROWS_PT)],
                        acc_out.at[r, c, pl.ds(row0, ROWS_PT)])
        if with_deg:
            @pl.when(c == 0)
            def _():
                pltpu.sync_copy(deg.at[pl.ds(t * DEG_PT, DEG_PT)],
                                deg_out.at[r, pl.ds(t * DEG_PT, DEG_PT)])

                @pl.when(t == 0)
                def _():
                    pltpu.sync_copy(deg.at[pl.ds(NS * DEG_PT, DEG_TAIL)],
                                    deg_out.at[r, pl.ds(NS * DEG_PT,
                                                        DEG_TAIL)])
        plsc.subcore_barrier()


def _make_sc_agg(width, k, with_deg, rel_stride):
    mesh = plsc.VectorSubcoreMesh(core_axis_name="c", subcore_axis_name="s",
                                  num_cores=NC, num_subcores=NS)
    out_type = [jax.ShapeDtypeStruct((3, NC, N, width), jnp.float32)]
    scratch = [
        pltpu.VMEM_SHARED((N, width), jnp.float32),
    ]
    if with_deg:
        out_type.append(jax.ShapeDtypeStruct((3, N), jnp.float32))
        scratch.append(pltpu.VMEM_SHARED((N,), jnp.float32))
    scratch += [pltpu.VMEM((k, SUB), jnp.int32)] * 6
    scratch += [pltpu.VMEM((k, SUB, width), jnp.float32)] * 3
    if with_deg:
        scratch.append(pltpu.VMEM((k, SUB), jnp.float32))
        scratch.append(pltpu.VMEM((DEG_PT,), jnp.float32))
    scratch += [pltpu.SemaphoreType.DMA] * 12
    return pl.kernel(
        functools.partial(_sc_agg_body, width, k, with_deg, rel_stride),
        out_type=out_type,
        mesh=mesh,
        scratch_types=scratch,
        compiler_params=pltpu.CompilerParams(use_tc_tiling_on_sc=False),
    )


def kernel(x, edge_index1, edge_index2, edge_index3,
           W1_1, b1_1, W2_1, b2_1, W3_1, b3_1,
           W1_2, b1_2, W2_2, b2_2, W3_2, b3_2,
           g1, be1, g2, be2, g3, be3):
    h_split = pl.pallas_call(
        _bn1_body,
        out_shape=jax.ShapeDtypeStruct((2 * N, 64), jnp.float32),
    )(x, g1.reshape(1, D_IN), be1.reshape(1, D_IN))

    e1 = edge_index1.reshape(2, NBLK, SUB)
    e2 = edge_index2.reshape(2, NBLK, SUB)
    e3 = edge_index3.reshape(2, NBLK, SUB)

    acc1, deg = _make_sc_agg(64, 5, True, 0)(h_split, e1, e2, e3)

    deg3 = deg.reshape(3, N, 1)
    w1 = jnp.stack([W1_1, W2_1, W3_1])
    b1 = jnp.stack([b1_1, b2_1, b3_1]).reshape(3, 1, D_H)
    w2 = jnp.stack([W1_2, W2_2, W3_2])

    acc_spec = lambda w: pl.BlockSpec((3, 2, RB, w), lambda i: (0, 0, i, 0))
    deg_spec = pl.BlockSpec((3, RB, 1), lambda i: (0, i, 0))
    full = lambda *s: pl.BlockSpec(s, lambda i: (0,) * len(s))
    row_spec = lambda w: pl.BlockSpec((RB, w), lambda i: (i, 0))

    h1, st1 = pl.pallas_call(
        _mid1_body,
        grid=(GRID,),
        in_specs=[acc_spec(64), deg_spec, full(3, D_IN, D_H),
                  full(3, 1, D_H)],
        out_specs=[row_spec(D_H), full(2, D_H)],
        out_shape=[jax.ShapeDtypeStruct((N, D_H), jnp.float32),
                   jax.ShapeDtypeStruct((2, D_H), jnp.float32)],
    )(acc1, deg3, w1, b1)

    table2 = pl.pallas_call(
        _mid2_body,
        grid=(GRID,),
        in_specs=[row_spec(D_H), full(2, D_H), full(3, D_H, D_OUT),
                  full(1, D_H), full(1, D_H)],
        out_specs=pl.BlockSpec((3, 2, RB, 32), lambda i: (0, 0, i, 0)),
        out_shape=jax.ShapeDtypeStruct((3, 2, N, 32), jnp.float32),
    )(h1, st1, w2, g2.reshape(1, D_H), be2.reshape(1, D_H))

    acc2 = _make_sc_agg(32, 5, False, 2 * N)(
        table2.reshape(3 * 2 * N, 32), e1, e2, e3)[0]

    b2 = jnp.stack([b1_2, b2_2, b3_2]).reshape(3, 1, D_OUT)
    h2, st2 = pl.pallas_call(
        _fin1_body,
        grid=(GRID,),
        in_specs=[acc_spec(32), deg_spec, full(3, 1, D_OUT)],
        out_specs=[row_spec(D_OUT), full(2, D_OUT)],
        out_shape=[jax.ShapeDtypeStruct((N, D_OUT), jnp.float32),
                   jax.ShapeDtypeStruct((2, D_OUT), jnp.float32)],
    )(acc2, deg3, b2)

    out = pl.pallas_call(
        _fin2_body,
        grid=(GRID,),
        in_specs=[row_spec(D_OUT), full(2, D_OUT), full(1, D_OUT),
                  full(1, D_OUT)],
        out_specs=row_spec(D_OUT),
        out_shape=jax.ShapeDtypeStruct((N, D_OUT), jnp.float32),
    )(h2, st2, g3.reshape(1, D_OUT), be3.reshape(1, D_OUT))
    return out


# speedup vs baseline: 12.8183x; 1.0113x over previous
import functools

import jax
import jax.numpy as jnp
from jax import lax
from jax.experimental import pallas as pl
from jax.experimental.pallas import tpu as pltpu
from jax.experimental.pallas import tpu_sc as plsc

N = 10000
E = 320000
D_IN = 128
D_H = 128
D_OUT = 64

NC = 2
NS = 16
SUB = 80
NBLK = E // SUB
NBT = NBLK // NS
ROWS_PT = N // NS
DEG_PT = 624
DEG_TAIL = N - NS * DEG_PT

_EPS = 1e-5


def _bn1_body(x_ref, g_ref, be_ref, out_ref):
    x = x_ref[...]
    mu = jnp.mean(x, axis=0, keepdims=True)
    xc = x - mu
    var = jnp.mean(xc * xc, axis=0, keepdims=True)
    hn = g_ref[...] * (xc * lax.rsqrt(var + _EPS)) + be_ref[...]
    out_ref[0:N, :] = hn[:, 0:64]
    out_ref[N:2 * N, :] = hn[:, 64:128]


RB = 2000
GRID = N // RB


def _mid1_body(acc_ref, deg_ref, w1_ref, b1_ref, h1_ref, st_ref):
    i = pl.program_id(0)
    h1 = None
    for r in range(3):
        hr = jnp.concatenate([acc_ref[r, 0], acc_ref[r, 1]], axis=1)
        hr = hr * (1.0 / (deg_ref[r] + 1.0))
        t = jnp.dot(hr, w1_ref[r], preferred_element_type=jnp.float32)
        t = t + b1_ref[r]
        h1 = t if h1 is None else h1 + t
    h1 = jnp.maximum(h1, 0.0)
    h1_ref[...] = h1
    part = jnp.concatenate(
        [jnp.sum(h1, axis=0, keepdims=True),
         jnp.sum(h1 * h1, axis=0, keepdims=True)], axis=0)

    @pl.when(i == 0)
    def _():
        st_ref[...] = part

    @pl.when(i != 0)
    def _():
        st_ref[...] = st_ref[...] + part


def _mid2_body(h1_ref, st_ref, w2_ref, g_ref, be_ref, out_ref):
    mu = st_ref[0:1, :] * (1.0 / N)
    var = st_ref[1:2, :] * (1.0 / N) - mu * mu
    hb = g_ref[...] * ((h1_ref[...] - mu) * lax.rsqrt(var + _EPS)) + be_ref[...]
    for r in range(3):
        y = jnp.dot(hb, w2_ref[r], preferred_element_type=jnp.float32)
        out_ref[r, 0] = y[:, 0:32]
        out_ref[r, 1] = y[:, 32:64]


def _fin1_body(acc_ref, deg_ref, b2_ref, h2_ref, st_ref):
    i = pl.program_id(0)
    h2 = None
    for r in range(3):
        hr = jnp.concatenate([acc_ref[r, 0], acc_ref[r, 1]], axis=1)
        t = hr * (1.0 / (deg_ref[r] + 1.0)) + b2_ref[r]
        h2 = t if h2 is None else h2 + t
    h2_ref[...] = h2
    part = jnp.concatenate(
        [jnp.sum(h2, axis=0, keepdims=True),
         jnp.sum(h2 * h2, axis=0, keepdims=True)], axis=0)

    @pl.when(i == 0)
    def _():
        st_ref[...] = part

    @pl.when(i != 0)
    def _():
        st_ref[...] = st_ref[...] + part


def _fin2_body(h2_ref, st_ref, g_ref, be_ref, out_ref):
    mu = st_ref[0:1, :] * (1.0 / N)
    var = st_ref[1:2, :] * (1.0 / N) - mu * mu
    hn = g_ref[...] * ((h2_ref[...] - mu) * lax.rsqrt(var + _EPS)) + be_ref[...]
    sg = 1.0 / (1.0 + jnp.exp(-hn))
    zmax = jnp.max(sg, axis=1, keepdims=True)
    zmin = jnp.min(sg, axis=1, keepdims=True)
    out = (sg - zmin) / (zmax - zmin)
    nrm = jnp.sqrt(jnp.sum(out * out, axis=1, keepdims=True))
    out_ref[...] = out / jnp.maximum(nrm, 1e-12)


def _sc_agg_body(width, k, with_deg, rel_stride, table_ref, e1_ref, e2_ref,
                 e3_ref, *rest):
    if with_deg:
        (acc_out, deg_out, acc, deg,
         src0, src1, src2, dst0, dst1, dst2, rows0, rows1, rows2,
         ones_v, zer_v, gs0, gs1, gs2, ss0, ss1, ss2,
         is0, is1, is2, id0, id1, id2) = rest
    else:
        (acc_out, acc,
         src0, src1, src2, dst0, dst1, dst2, rows0, rows1, rows2,
         gs0, gs1, gs2, ss0, ss1, ss2,
         is0, is1, is2, id0, id1, id2) = rest
        deg = None
    srcs = (src0, src1, src2)
    dsts = (dst0, dst1, dst2)
    rows = (rows0, rows1, rows2)
    gss = (gs0, gs1, gs2)
    sss = (ss0, ss1, ss2)
    iss = (is0, is1, is2)
    ids = (id0, id1, id2)

    c = lax.axis_index("c")
    t = lax.axis_index("s")
    row0 = t * ROWS_PT
    cN = c * N

    if with_deg:
        for j in range(k):
            for m in range(SUB // 16):
                ones_v[j, pl.ds(m * 16, 16)] = jnp.ones((16,), jnp.float32)
        for j in range(DEG_PT // 16):
            zer_v[pl.ds(j * 16, 16)] = jnp.zeros((16,), jnp.float32)

    nphase = NBT // k
    ngrp = (nphase - 2) // 3
    base = t * NBT

    def fire_src(e_ref, b, blk):
        pltpu.async_copy(e_ref.at[0, pl.ds(blk, k)], srcs[b], iss[b])

    def phase_fire(e_ref, bias, b, blk):
        pltpu.make_async_copy(e_ref.at[0, pl.ds(blk, k)], srcs[b],
                              iss[b]).wait()
        for j in range(k):
            for m in range(SUB // 16):
                sl = pl.ds(m * 16, 16)
                srcs[b][j, sl] = srcs[b][j, sl] + bias
        pltpu.async_copy(e_ref.at[1, pl.ds(blk, k)], dsts[b], ids[b])
        for j in range(k):
            pltpu.async_copy(table_ref.at[srcs[b].at[j]], rows[b].at[j],
                             gss[b])

    def phase_complete(e_ref, acc, deg, b, blk):
        for j in range(k):
            pltpu.make_async_copy(table_ref.at[srcs[b].at[j]],
                                  rows[b].at[j], gss[b]).wait()
        pltpu.make_async_copy(e_ref.at[1, pl.ds(blk, k)], dsts[b],
                              ids[b]).wait()
        for j in range(k):
            pltpu.async_copy(rows[b].at[j], acc.at[dsts[b].at[j]],
                             sss[b], add=True)
        if with_deg:
            @pl.when(c == 0)
            def _():
                for j in range(k):
                    pltpu.async_copy(ones_v.at[j], deg.at[dsts[b].at[j]],
                                     sss[b], add=True)

    def drain_scatters(acc, deg, b):
        for j in range(k):
            pltpu.make_async_copy(rows[b].at[j], acc.at[dsts[b].at[j]],
                                  sss[b]).wait()
        if with_deg:
            @pl.when(c == 0)
            def _():
                for j in range(k):
                    pltpu.make_async_copy(ones_v.at[j], deg.at[dsts[b].at[j]],
                                          sss[b]).wait()

    for r, e_ref in enumerate((e1_ref, e2_ref, e3_ref)):
        bias = r * rel_stride + cN
        pltpu.sync_copy(table_ref.at[pl.ds(r * rel_stride + cN + row0,
                                           ROWS_PT)],
                        acc.at[pl.ds(row0, ROWS_PT)])
        if with_deg:
            @pl.when(c == 0)
            def _():
                pltpu.sync_copy(zer_v, deg.at[pl.ds(t * DEG_PT, DEG_PT)])

                @pl.when(t == 0)
                def _():
                    pltpu.sync_copy(zer_v.at[pl.ds(0, DEG_TAIL)],
                                    deg.at[pl.ds(NS * DEG_PT, DEG_TAIL)])
        plsc.subcore_barrier()

        fire_src(e_ref, 0, base)
        fire_src(e_ref, 1, base + k)
        phase_fire(e_ref, bias, 0, base)
        fire_src(e_ref, 2, base + 2 * k)
        phase_fire(e_ref, bias, 1, base + k)
        phase_complete(e_ref, acc, deg, 0, base)

        def ring_body(g, _, e_ref=e_ref, bias=bias):
            b0 = base + (2 + 3 * g) * k
            @pl.when(g >= 1)
            def _():
                drain_scatters(acc, deg, 2)
            phase_fire(e_ref, bias, 2, b0)
            fire_src(e_ref, 0, b0 + k)
            phase_complete(e_ref, acc, deg, 1, b0 - k)
            drain_scatters(acc, deg, 0)
            phase_fire(e_ref, bias, 0, b0 + k)
            fire_src(e_ref, 1, b0 + 2 * k)
            phase_complete(e_ref, acc, deg, 2, b0)
            drain_scatters(acc, deg, 1)
            phase_fire(e_ref, bias, 1, b0 + 2 * k)

            @pl.when(g < ngrp - 1)
            def _():
                fire_src(e_ref, 2, b0 + 3 * k)
            phase_complete(e_ref, acc, deg, 0, b0 + k)
            return 0

        lax.fori_loop(0, ngrp, ring_body, 0)
        phase_complete(e_ref, acc, deg, 1, base + (nphase - 1) * k)
        drain_scatters(acc, deg, 2)
        drain_scatters(acc, deg, 0)
        drain_scatters(acc, deg, 1)
        plsc.subcore_barrier()

        pltpu.sync_copy(acc.at[pl.ds(row0, ROWS_PT)],
                        acc_out.at[r, c, pl.ds(row0, ROWS_PT)])
        if with_deg:
            @pl.when(c == 0)
            def _():
                pltpu.sync_copy(deg.at[pl.ds(t * DEG_PT, DEG_PT)],
                                deg_out.at[r, pl.ds(t * DEG_PT, DEG_PT)])

                @pl.when(t == 0)
                def _():
                    pltpu.sync_copy(deg.at[pl.ds(NS * DEG_PT, DEG_TAIL)],
                                    deg_out.at[r, pl.ds(NS * DEG_PT,
                                                        DEG_TAIL)])
        plsc.subcore_barrier()


def _make_sc_agg(width, k, with_deg, rel_stride):
    mesh = plsc.VectorSubcoreMesh(core_axis_name="c", subcore_axis_name="s",
                                  num_cores=NC, num_subcores=NS)
    out_type = [jax.ShapeDtypeStruct((3, NC, N, width), jnp.float32)]
    scratch = [
        pltpu.VMEM_SHARED((N, width), jnp.float32),
    ]
    if with_deg:
        out_type.append(jax.ShapeDtypeStruct((3, N), jnp.float32))
        scratch.append(pltpu.VMEM_SHARED((N,), jnp.float32))
    scratch += [pltpu.VMEM((k, SUB), jnp.int32)] * 6
    scratch += [pltpu.VMEM((k, SUB, width), jnp.float32)] * 3
    if with_deg:
        scratch.append(pltpu.VMEM((k, SUB), jnp.float32))
        scratch.append(pltpu.VMEM((DEG_PT,), jnp.float32))
    scratch += [pltpu.SemaphoreType.DMA] * 12
    return pl.kernel(
        functools.partial(_sc_agg_body, width, k, with_deg, rel_stride),
        out_type=out_type,
        mesh=mesh,
        scratch_types=scratch,
        compiler_params=pltpu.CompilerParams(use_tc_tiling_on_sc=False),
    )


def kernel(x, edge_index1, edge_index2, edge_index3,
           W1_1, b1_1, W2_1, b2_1, W3_1, b3_1,
           W1_2, b1_2, W2_2, b2_2, W3_2, b3_2,
           g1, be1, g2, be2, g3, be3):
    h_split = pl.pallas_call(
        _bn1_body,
        out_shape=jax.ShapeDtypeStruct((2 * N, 64), jnp.float32),
    )(x, g1.reshape(1, D_IN), be1.reshape(1, D_IN))

    e1 = edge_index1.reshape(2, NBLK, SUB)
    e2 = edge_index2.reshape(2, NBLK, SUB)
    e3 = edge_index3.reshape(2, NBLK, SUB)

    acc1, deg = _make_sc_agg(64, 5, True, 0)(h_split, e1, e2, e3)

    deg3 = deg.reshape(3, N, 1)
    w1 = jnp.stack([W1_1, W2_1, W3_1])
    b1 = jnp.stack([b1_1, b2_1, b3_1]).reshape(3, 1, D_H)
    w2 = jnp.stack([W1_2, W2_2, W3_2])

    acc_spec = lambda w: pl.BlockSpec((3, 2, RB, w), lambda i: (0, 0, i, 0))
    deg_spec = pl.BlockSpec((3, RB, 1), lambda i: (0, i, 0))
    full = lambda *s: pl.BlockSpec(s, lambda i: (0,) * len(s))
    row_spec = lambda w: pl.BlockSpec((RB, w), lambda i: (i, 0))

    h1, st1 = pl.pallas_call(
        _mid1_body,
        grid=(GRID,),
        in_specs=[acc_spec(64), deg_spec, full(3, D_IN, D_H),
                  full(3, 1, D_H)],
        out_specs=[row_spec(D_H), full(2, D_H)],
        out_shape=[jax.ShapeDtypeStruct((N, D_H), jnp.float32),
                   jax.ShapeDtypeStruct((2, D_H), jnp.float32)],
    )(acc1, deg3, w1, b1)

    table2 = pl.pallas_call(
        _mid2_body,
        grid=(GRID,),
        in_specs=[row_spec(D_H), full(2, D_H), full(3, D_H, D_OUT),
                  full(1, D_H), full(1, D_H)],
        out_specs=pl.BlockSpec((3, 2, RB, 32), lambda i: (0, 0, i, 0)),
        out_shape=jax.ShapeDtypeStruct((3, 2, N, 32), jnp.float32),
    )(h1, st1, w2, g2.reshape(1, D_H), be2.reshape(1, D_H))

    acc2 = _make_sc_agg(32, 5, False, 2 * N)(
        table2.reshape(3 * 2 * N, 32), e1, e2, e3)[0]

    b2 = jnp.stack([b1_2, b2_2, b3_2]).reshape(3, 1, D_OUT)
    h2, st2 = pl.pallas_call(
        _fin1_body,
        grid=(GRID,),
        in_specs=[acc_spec(32), deg_spec, full(3, 1, D_OUT)],
        out_specs=[row_spec(D_OUT), full(2, D_OUT)],
        out_shape=[jax.ShapeDtypeStruct((N, D_OUT), jnp.float32),
                   jax.ShapeDtypeStruct((2, D_OUT), jnp.float32)],
    )(acc2, deg3, b2)

    out = pl.pallas_call(
        _fin2_body,
        grid=(GRID,),
        in_specs=[row_spec(D_OUT), full(2, D_OUT), full(1, D_OUT),
                  full(1, D_OUT)],
        out_specs=row_spec(D_OUT),
        out_shape=jax.ShapeDtypeStruct((N, D_OUT), jnp.float32),
    )(h2, st2, g3.reshape(1, D_OUT), be3.reshape(1, D_OUT))
    return out


# speedup vs baseline: 12.8901x; 1.0056x over previous
import functools

import jax
import jax.numpy as jnp
from jax import lax
from jax.experimental import pallas as pl
from jax.experimental.pallas import tpu as pltpu
from jax.experimental.pallas import tpu_sc as plsc

N = 10000
E = 320000
D_IN = 128
D_H = 128
D_OUT = 64

NC = 2
NS = 16
SUB = 80
NBLK = E // SUB
NBT = NBLK // NS
ROWS_PT = N // NS
DEG_PT = 624
DEG_TAIL = N - NS * DEG_PT

_EPS = 1e-5


def _bn1_body(x_ref, g_ref, be_ref, out_ref):
    x = x_ref[...]
    mu = jnp.mean(x, axis=0, keepdims=True)
    xc = x - mu
    var = jnp.mean(xc * xc, axis=0, keepdims=True)
    hn = g_ref[...] * (xc * lax.rsqrt(var + _EPS)) + be_ref[...]
    out_ref[0:N, :] = hn[:, 0:64]
    out_ref[N:2 * N, :] = hn[:, 64:128]


RB = 2000
GRID = N // RB


def _mid1_body(acc_ref, deg_ref, w1_ref, b1_ref, h1_ref, st_ref):
    i = pl.program_id(0)
    h1 = None
    for r in range(3):
        hr = jnp.concatenate([acc_ref[r, 0], acc_ref[r, 1]], axis=1)
        hr = hr * (1.0 / (deg_ref[r] + 1.0))
        t = jnp.dot(hr, w1_ref[r], preferred_element_type=jnp.float32)
        t = t + b1_ref[r]
        h1 = t if h1 is None else h1 + t
    h1 = jnp.maximum(h1, 0.0)
    h1_ref[...] = h1
    part = jnp.concatenate(
        [jnp.sum(h1, axis=0, keepdims=True),
         jnp.sum(h1 * h1, axis=0, keepdims=True)], axis=0)

    @pl.when(i == 0)
    def _():
        st_ref[...] = part

    @pl.when(i != 0)
    def _():
        st_ref[...] = st_ref[...] + part


def _mid2_body(h1_ref, st_ref, w2_ref, g_ref, be_ref, out_ref):
    mu = st_ref[0:1, :] * (1.0 / N)
    var = st_ref[1:2, :] * (1.0 / N) - mu * mu
    hb = g_ref[...] * ((h1_ref[...] - mu) * lax.rsqrt(var + _EPS)) + be_ref[...]
    for r in range(3):
        y = jnp.dot(hb, w2_ref[r], preferred_element_type=jnp.float32)
        out_ref[r, 0] = y[:, 0:32]
        out_ref[r, 1] = y[:, 32:64]


def _fin1_body(acc_ref, deg_ref, b2_ref, h2_ref, st_ref):
    i = pl.program_id(0)
    h2 = None
    for r in range(3):
        hr = jnp.concatenate([acc_ref[r, 0], acc_ref[r, 1]], axis=1)
        t = hr * (1.0 / (deg_ref[r] + 1.0)) + b2_ref[r]
        h2 = t if h2 is None else h2 + t
    h2_ref[...] = h2
    part = jnp.concatenate(
        [jnp.sum(h2, axis=0, keepdims=True),
         jnp.sum(h2 * h2, axis=0, keepdims=True)], axis=0)

    @pl.when(i == 0)
    def _():
        st_ref[...] = part

    @pl.when(i != 0)
    def _():
        st_ref[...] = st_ref[...] + part


def _fin2_body(h2_ref, st_ref, g_ref, be_ref, out_ref):
    mu = st_ref[0:1, :] * (1.0 / N)
    var = st_ref[1:2, :] * (1.0 / N) - mu * mu
    hn = g_ref[...] * ((h2_ref[...] - mu) * lax.rsqrt(var + _EPS)) + be_ref[...]
    sg = 1.0 / (1.0 + jnp.exp(-hn))
    zmax = jnp.max(sg, axis=1, keepdims=True)
    zmin = jnp.min(sg, axis=1, keepdims=True)
    out = (sg - zmin) / (zmax - zmin)
    nrm = jnp.sqrt(jnp.sum(out * out, axis=1, keepdims=True))
    out_ref[...] = out / jnp.maximum(nrm, 1e-12)


def _sc_agg_body(width, k, with_deg, rel_stride, table_ref, e1_ref, e2_ref,
                 e3_ref, *rest):
    if with_deg:
        (acc_out, deg_out, acc, deg,
         src0, src1, src2, dst0, dst1, dst2, rows0, rows1, rows2,
         ones_v, zer_v, gs0, gs1, gs2, ss0, ss1, ss2,
         is0, is1, is2, id0, id1, id2) = rest
    else:
        (acc_out, acc,
         src0, src1, src2, dst0, dst1, dst2, rows0, rows1, rows2,
         gs0, gs1, gs2, ss0, ss1, ss2,
         is0, is1, is2, id0, id1, id2) = rest
        deg = None
    srcs = (src0, src1, src2)
    dsts = (dst0, dst1, dst2)
    rows = (rows0, rows1, rows2)
    gss = (gs0, gs1, gs2)
    sss = (ss0, ss1, ss2)
    iss = (is0, is1, is2)
    ids = (id0, id1, id2)

    c = lax.axis_index("c")
    t = lax.axis_index("s")
    row0 = t * ROWS_PT
    cN = c * N

    if with_deg:
        for j in range(k):
            for m in range(SUB // 16):
                ones_v[j, pl.ds(m * 16, 16)] = jnp.ones((16,), jnp.float32)
        for j in range(DEG_PT // 16):
            zer_v[pl.ds(j * 16, 16)] = jnp.zeros((16,), jnp.float32)

    deg_core = [0]
    nphase = NBT // k
    ngrp = (nphase - 2) // 3
    base = t * NBT

    def fire_src(e_ref, b, blk):
        pltpu.async_copy(e_ref.at[0, pl.ds(blk, k)], srcs[b], iss[b])

    def phase_fire(e_ref, bias, b, blk):
        pltpu.make_async_copy(e_ref.at[0, pl.ds(blk, k)], srcs[b],
                              iss[b]).wait()
        for j in range(k):
            for m in range(SUB // 16):
                sl = pl.ds(m * 16, 16)
                srcs[b][j, sl] = srcs[b][j, sl] + bias
        pltpu.async_copy(e_ref.at[1, pl.ds(blk, k)], dsts[b], ids[b])
        for j in range(k):
            pltpu.async_copy(table_ref.at[srcs[b].at[j]], rows[b].at[j],
                             gss[b])

    def phase_complete(e_ref, acc, deg, b, blk):
        for j in range(k):
            pltpu.make_async_copy(table_ref.at[srcs[b].at[j]],
                                  rows[b].at[j], gss[b]).wait()
        pltpu.make_async_copy(e_ref.at[1, pl.ds(blk, k)], dsts[b],
                              ids[b]).wait()
        for j in range(k):
            pltpu.async_copy(rows[b].at[j], acc.at[dsts[b].at[j]],
                             sss[b], add=True)
        if with_deg:
            @pl.when(c == deg_core[0])
            def _():
                for j in range(k):
                    pltpu.async_copy(ones_v.at[j], deg.at[dsts[b].at[j]],
                                     sss[b], add=True)

    def drain_scatters(acc, deg, b):
        for j in range(k):
            pltpu.make_async_copy(rows[b].at[j], acc.at[dsts[b].at[j]],
                                  sss[b]).wait()
        if with_deg:
            @pl.when(c == deg_core[0])
            def _():
                for j in range(k):
                    pltpu.make_async_copy(ones_v.at[j], deg.at[dsts[b].at[j]],
                                          sss[b]).wait()

    for r, e_ref in enumerate((e1_ref, e2_ref, e3_ref)):
        bias = r * rel_stride + cN
        deg_core[0] = r % 2
        pltpu.sync_copy(table_ref.at[pl.ds(r * rel_stride + cN + row0,
                                           ROWS_PT)],
                        acc.at[pl.ds(row0, ROWS_PT)])
        if with_deg:
            @pl.when(c == r % 2)
            def _():
                pltpu.sync_copy(zer_v, deg.at[pl.ds(t * DEG_PT, DEG_PT)])

                @pl.when(t == 0)
                def _():
                    pltpu.sync_copy(zer_v.at[pl.ds(0, DEG_TAIL)],
                                    deg.at[pl.ds(NS * DEG_PT, DEG_TAIL)])
        plsc.subcore_barrier()

        fire_src(e_ref, 0, base)
        fire_src(e_ref, 1, base + k)
        phase_fire(e_ref, bias, 0, base)
        fire_src(e_ref, 2, base + 2 * k)
        phase_fire(e_ref, bias, 1, base + k)
        phase_complete(e_ref, acc, deg, 0, base)

        def ring_body(g, _, e_ref=e_ref, bias=bias):
            b0 = base + (2 + 3 * g) * k
            @pl.when(g >= 1)
            def _():
                drain_scatters(acc, deg, 2)
            phase_fire(e_ref, bias, 2, b0)
            fire_src(e_ref, 0, b0 + k)
            phase_complete(e_ref, acc, deg, 1, b0 - k)
            drain_scatters(acc, deg, 0)
            phase_fire(e_ref, bias, 0, b0 + k)
            fire_src(e_ref, 1, b0 + 2 * k)
            phase_complete(e_ref, acc, deg, 2, b0)
            drain_scatters(acc, deg, 1)
            phase_fire(e_ref, bias, 1, b0 + 2 * k)

            @pl.when(g < ngrp - 1)
            def _():
                fire_src(e_ref, 2, b0 + 3 * k)
            phase_complete(e_ref, acc, deg, 0, b0 + k)
            return 0

        lax.fori_loop(0, ngrp, ring_body, 0)
        phase_complete(e_ref, acc, deg, 1, base + (nphase - 1) * k)
        drain_scatters(acc, deg, 2)
        drain_scatters(acc, deg, 0)
        drain_scatters(acc, deg, 1)
        plsc.subcore_barrier()

        pltpu.sync_copy(acc.at[pl.ds(row0, ROWS_PT)],
                        acc_out.at[r, c, pl.ds(row0, ROWS_PT)])
        if with_deg:
            @pl.when(c == r % 2)
            def _():
                pltpu.sync_copy(deg.at[pl.ds(t * DEG_PT, DEG_PT)],
                                deg_out.at[r, pl.ds(t * DEG_PT, DEG_PT)])

                @pl.when(t == 0)
                def _():
                    pltpu.sync_copy(deg.at[pl.ds(NS * DEG_PT, DEG_TAIL)],
                                    deg_out.at[r, pl.ds(NS * DEG_PT,
                                                        DEG_TAIL)])
        plsc.subcore_barrier()


def _make_sc_agg(width, k, with_deg, rel_stride):
    mesh = plsc.VectorSubcoreMesh(core_axis_name="c", subcore_axis_name="s",
                                  num_cores=NC, num_subcores=NS)
    out_type = [jax.ShapeDtypeStruct((3, NC, N, width), jnp.float32)]
    scratch = [
        pltpu.VMEM_SHARED((N, width), jnp.float32),
    ]
    if with_deg:
        out_type.append(jax.ShapeDtypeStruct((3, N), jnp.float32))
        scratch.append(pltpu.VMEM_SHARED((N,), jnp.float32))
    scratch += [pltpu.VMEM((k, SUB), jnp.int32)] * 6
    scratch += [pltpu.VMEM((k, SUB, width), jnp.float32)] * 3
    if with_deg:
        scratch.append(pltpu.VMEM((k, SUB), jnp.float32))
        scratch.append(pltpu.VMEM((DEG_PT,), jnp.float32))
    scratch += [pltpu.SemaphoreType.DMA] * 12
    return pl.kernel(
        functools.partial(_sc_agg_body, width, k, with_deg, rel_stride),
        out_type=out_type,
        mesh=mesh,
        scratch_types=scratch,
        compiler_params=pltpu.CompilerParams(use_tc_tiling_on_sc=False),
    )


def kernel(x, edge_index1, edge_index2, edge_index3,
           W1_1, b1_1, W2_1, b2_1, W3_1, b3_1,
           W1_2, b1_2, W2_2, b2_2, W3_2, b3_2,
           g1, be1, g2, be2, g3, be3):
    h_split = pl.pallas_call(
        _bn1_body,
        out_shape=jax.ShapeDtypeStruct((2 * N, 64), jnp.float32),
    )(x, g1.reshape(1, D_IN), be1.reshape(1, D_IN))

    e1 = edge_index1.reshape(2, NBLK, SUB)
    e2 = edge_index2.reshape(2, NBLK, SUB)
    e3 = edge_index3.reshape(2, NBLK, SUB)

    acc1, deg = _make_sc_agg(64, 5, True, 0)(h_split, e1, e2, e3)

    deg3 = deg.reshape(3, N, 1)
    w1 = jnp.stack([W1_1, W2_1, W3_1])
    b1 = jnp.stack([b1_1, b2_1, b3_1]).reshape(3, 1, D_H)
    w2 = jnp.stack([W1_2, W2_2, W3_2])

    acc_spec = lambda w: pl.BlockSpec((3, 2, RB, w), lambda i: (0, 0, i, 0))
    deg_spec = pl.BlockSpec((3, RB, 1), lambda i: (0, i, 0))
    full = lambda *s: pl.BlockSpec(s, lambda i: (0,) * len(s))
    row_spec = lambda w: pl.BlockSpec((RB, w), lambda i: (i, 0))

    h1, st1 = pl.pallas_call(
        _mid1_body,
        grid=(GRID,),
        in_specs=[acc_spec(64), deg_spec, full(3, D_IN, D_H),
                  full(3, 1, D_H)],
        out_specs=[row_spec(D_H), full(2, D_H)],
        out_shape=[jax.ShapeDtypeStruct((N, D_H), jnp.float32),
                   jax.ShapeDtypeStruct((2, D_H), jnp.float32)],
    )(acc1, deg3, w1, b1)

    table2 = pl.pallas_call(
        _mid2_body,
        grid=(GRID,),
        in_specs=[row_spec(D_H), full(2, D_H), full(3, D_H, D_OUT),
                  full(1, D_H), full(1, D_H)],
        out_specs=pl.BlockSpec((3, 2, RB, 32), lambda i: (0, 0, i, 0)),
        out_shape=jax.ShapeDtypeStruct((3, 2, N, 32), jnp.float32),
    )(h1, st1, w2, g2.reshape(1, D_H), be2.reshape(1, D_H))

    acc2 = _make_sc_agg(32, 5, False, 2 * N)(
        table2.reshape(3 * 2 * N, 32), e1, e2, e3)[0]

    b2 = jnp.stack([b1_2, b2_2, b3_2]).reshape(3, 1, D_OUT)
    h2, st2 = pl.pallas_call(
        _fin1_body,
        grid=(GRID,),
        in_specs=[acc_spec(32), deg_spec, full(3, 1, D_OUT)],
        out_specs=[row_spec(D_OUT), full(2, D_OUT)],
        out_shape=[jax.ShapeDtypeStruct((N, D_OUT), jnp.float32),
                   jax.ShapeDtypeStruct((2, D_OUT), jnp.float32)],
    )(acc2, deg3, b2)

    out = pl.pallas_call(
        _fin2_body,
        grid=(GRID,),
        in_specs=[row_spec(D_OUT), full(2, D_OUT), full(1, D_OUT),
                  full(1, D_OUT)],
        out_specs=row_spec(D_OUT),
        out_shape=jax.ShapeDtypeStruct((N, D_OUT), jnp.float32),
    )(h2, st2, g3.reshape(1, D_OUT), be3.reshape(1, D_OUT))
    return out
